# Initial kernel scaffold; baseline (speedup 1.0000x reference)
import jax
import jax.numpy as jnp
from jax.experimental import pallas as pl


def kernel(input, adj, W, b):
    raise NotImplementedError("write your pallas kernel here")


# speedup vs baseline: 5.3986x; 5.3986x over previous
import functools

import jax
import jax.numpy as jnp
from jax import lax
from jax.experimental import pallas as pl
from jax.experimental.pallas import tpu as pltpu
from jax.experimental.pallas import tpu_sc as plsc

NC = 2
NS = 16
NW = NC * NS
LANES = 16


def _matmul(x, w):
    m, k = x.shape
    n = w.shape[1]
    bm = 1000

    def body(x_ref, w_ref, o_ref):
        o_ref[...] = jnp.dot(x_ref[...], w_ref[...],
                             preferred_element_type=jnp.float32)

    return pl.pallas_call(
        body,
        grid=(m // bm,),
        in_specs=[pl.BlockSpec((bm, k), lambda i: (i, 0)),
                  pl.BlockSpec((k, n), lambda i: (0, 0))],
        out_specs=pl.BlockSpec((bm, n), lambda i: (i, 0)),
        out_shape=jax.ShapeDtypeStruct((m, n), jnp.float32),
    )(x, w)


def _combine(partials, bias, n_nodes):
    d = partials.shape[2]
    bm = 1000

    def body(p_ref, b_ref, o_ref):
        o_ref[...] = p_ref[0] + p_ref[1] + b_ref[...]

    return pl.pallas_call(
        body,
        grid=(n_nodes // bm,),
        in_specs=[pl.BlockSpec((2, bm, d), lambda i: (0, i, 0)),
                  pl.BlockSpec((1, d), lambda i: (0, 0))],
        out_specs=pl.BlockSpec((bm, d), lambda i: (i, 0)),
        out_shape=jax.ShapeDtypeStruct((n_nodes, d), jnp.float32),
    )(partials, bias)


def _sc_aggregate(support, src, dst):
    n_nodes, d = support.shape
    n_edges = src.shape[0]
    epw = n_edges // NW
    chunk = 80
    nchunk = epw // chunk
    rpt = -(-n_nodes // (NS * 8)) * 8
    n_pad = rpt * NS

    mesh = plsc.VectorSubcoreMesh(core_axis_name="c", subcore_axis_name="s",
                                  num_cores=NC, num_subcores=NS)

    @functools.partial(
        pl.kernel,
        out_type=jax.ShapeDtypeStruct((NC * n_pad, d), jnp.float32),
        mesh=mesh,
        scratch_types=[
            pltpu.VMEM_SHARED((n_pad, d), jnp.float32),
            pltpu.VMEM((chunk,), jnp.int32),
            pltpu.VMEM((chunk,), jnp.int32),
            pltpu.VMEM((chunk, d), jnp.float32),
            pltpu.SemaphoreType.DMA,
        ],
    )
    def agg(sup_hbm, src_hbm, dst_hbm, out_hbm, acc, idx_s, idx_d, rows, sem):
        c = lax.axis_index("c")
        s = lax.axis_index("s")

        zeros = jnp.zeros((LANES,), jnp.float32)
        per_row = d // LANES

        def zstep(i, _):
            rows[i // per_row, pl.ds((i % per_row) * LANES, LANES)] = zeros
            return 0

        lax.fori_loop(0, chunk * per_row, zstep, 0)
        off_r = 0
        while off_r + chunk <= rpt:
            pltpu.sync_copy(rows, acc.at[pl.ds(s * rpt + off_r, chunk)])
            off_r += chunk
        if rpt - off_r:
            pltpu.sync_copy(rows.at[pl.ds(0, rpt - off_r)],
                            acc.at[pl.ds(s * rpt + off_r, rpt - off_r)])
        plsc.subcore_barrier()

        base = (c * NS + s) * epw

        def step(i, _):
            off = base + i * chunk
            pltpu.sync_copy(src_hbm.at[pl.ds(off, chunk)], idx_s)
            pltpu.sync_copy(dst_hbm.at[pl.ds(off, chunk)], idx_d)
            pltpu.async_copy(sup_hbm.at[idx_s], rows, sem).wait()
            pltpu.sync_copy(rows, acc.at[idx_d], add=True)
            return 0

        lax.fori_loop(0, nchunk, step, 0)
        plsc.subcore_barrier()

        pltpu.sync_copy(acc.at[pl.ds(s * rpt, rpt)],
                        out_hbm.at[pl.ds(c * n_pad + s * rpt, rpt)])

    return agg(support, src, dst)


def kernel(input, adj, W, b):
    n_nodes, d_in = input.shape
    d_out = W.shape[1]
    support = _matmul(input, W)
    partials = _sc_aggregate(support, adj[0], adj[1])
    partials = partials.reshape(NC, partials.shape[0] // NC, d_out)
    return _combine(partials, b.reshape(1, d_out), n_nodes)


# speedup vs baseline: 12.3426x; 2.2863x over previous
import functools

import jax
import jax.numpy as jnp
from jax import lax
from jax.experimental import pallas as pl
from jax.experimental.pallas import tpu as pltpu
from jax.experimental.pallas import tpu_sc as plsc

NC = 2
NS = 16
NW = NC * NS
LANES = 16


def _matmul(x, w):
    m, k = x.shape
    n = w.shape[1]
    bm = 1000

    def body(x_ref, w_ref, o_ref):
        o_ref[...] = jnp.dot(x_ref[...], w_ref[...],
                             preferred_element_type=jnp.float32)

    return pl.pallas_call(
        body,
        grid=(m // bm,),
        in_specs=[pl.BlockSpec((bm, k), lambda i: (i, 0)),
                  pl.BlockSpec((k, n), lambda i: (0, 0))],
        out_specs=pl.BlockSpec((bm, n), lambda i: (i, 0)),
        out_shape=jax.ShapeDtypeStruct((m, n), jnp.float32),
    )(x, w)


def _combine(partials, bias, n_nodes):
    d = partials.shape[2]
    bm = 1000

    def body(p_ref, b_ref, o_ref):
        o_ref[...] = p_ref[0] + p_ref[1] + b_ref[...]

    return pl.pallas_call(
        body,
        grid=(n_nodes // bm,),
        in_specs=[pl.BlockSpec((2, bm, d), lambda i: (0, i, 0)),
                  pl.BlockSpec((1, d), lambda i: (0, 0))],
        out_specs=pl.BlockSpec((bm, d), lambda i: (i, 0)),
        out_shape=jax.ShapeDtypeStruct((n_nodes, d), jnp.float32),
    )(partials, bias)


def _sc_aggregate(support, adj1, chunk):
    n_nodes, d = support.shape
    n_edges = adj1.shape[0] // 2
    epw = n_edges // NW
    nchunk = epw // chunk
    rpt = -(-n_nodes // (NS * 8)) * 8
    n_pad = rpt * NS

    mesh = plsc.VectorSubcoreMesh(core_axis_name="c", subcore_axis_name="s",
                                  num_cores=NC, num_subcores=NS)

    @functools.partial(
        pl.kernel,
        out_type=jax.ShapeDtypeStruct((NC * n_pad, d), jnp.float32),
        mesh=mesh,
        scratch_types=[
            pltpu.VMEM_SHARED((n_pad, d), jnp.float32),
            pltpu.VMEM((epw,), jnp.int32),
            pltpu.VMEM((chunk,), jnp.int32),
            pltpu.VMEM((chunk,), jnp.int32),
            pltpu.VMEM((chunk, d), jnp.float32),
            pltpu.VMEM((chunk, d), jnp.float32),
            pltpu.SemaphoreType.DMA,
            pltpu.SemaphoreType.DMA,
            pltpu.SemaphoreType.DMA,
            pltpu.SemaphoreType.DMA,
            pltpu.SemaphoreType.DMA,
            pltpu.SemaphoreType.DMA,
        ],
    )
    def agg(sup_hbm, adj_hbm, out_hbm, acc, src_all, idxd0, idxd1, rows0,
            rows1, gsem0, gsem1, dsem0, dsem1, ssem0, ssem1):
        c = lax.axis_index("c")
        s = lax.axis_index("s")
        w = c * NS + s
        rows = (rows0, rows1)
        idxd = (idxd0, idxd1)
        gsem = (gsem0, gsem1)
        dsem = (dsem0, dsem1)
        ssem = (ssem0, ssem1)

        pltpu.sync_copy(adj_hbm.at[pl.ds(w * epw, epw)], src_all)
        dst_base = n_edges + w * epw

        zeros = jnp.zeros((LANES,), jnp.float32)
        per_row = d // LANES

        def zstep(i, _):
            rows0[i // per_row, pl.ds((i % per_row) * LANES, LANES)] = zeros
            return 0

        lax.fori_loop(0, chunk * per_row, zstep, 0)
        off_r = 0
        while off_r + chunk <= rpt:
            pltpu.sync_copy(rows0, acc.at[pl.ds(s * rpt + off_r, chunk)])
            off_r += chunk
        if rpt - off_r:
            pltpu.sync_copy(rows0.at[pl.ds(0, rpt - off_r)],
                            acc.at[pl.ds(s * rpt + off_r, rpt - off_r)])
        plsc.subcore_barrier()

        def start_g(i, b):
            pltpu.async_copy(sup_hbm.at[src_all.at[pl.ds(i * chunk, chunk)]],
                             rows[b], gsem[b])

        def wait_g(b):
            pltpu.make_async_copy(sup_hbm.at[src_all.at[pl.ds(0, chunk)]],
                                  rows[b], gsem[b]).wait()

        def start_d(i, b):
            pltpu.async_copy(adj_hbm.at[pl.ds(dst_base + i * chunk, chunk)],
                             idxd[b], dsem[b])

        def wait_d(b):
            pltpu.make_async_copy(adj_hbm.at[pl.ds(dst_base, chunk)],
                                  idxd[b], dsem[b]).wait()

        def scat(b):
            pltpu.async_copy(rows[b], acc.at[idxd[b]], ssem[b],
                             add=True).wait()

        for b in range(2):
            start_g(b, b)
            start_d(b, b)
        n_steady = (nchunk - 3) // 2

        def step(j, _):
            for b in range(2):
                i = 2 * j + b
                wait_g(b)
                wait_d(b)
                scat(b)
                start_g(i + 2, b)
                start_d(i + 2, b)
            return 0

        lax.fori_loop(0, n_steady, step, 0)
        for i in range(2 * n_steady, nchunk):
            b = i % 2
            wait_g(b)
            wait_d(b)
            scat(b)
            if i + 2 < nchunk:
                start_g(i + 2, b)
                start_d(i + 2, b)
        plsc.subcore_barrier()

        pltpu.sync_copy(acc.at[pl.ds(s * rpt, rpt)],
                        out_hbm.at[pl.ds(c * n_pad + s * rpt, rpt)])

    return agg(support, adj1)


def kernel(input, adj, W, b):
    n_nodes, d_in = input.shape
    d_out = W.shape[1]
    support = _matmul(input, W)
    partials = _sc_aggregate(support, adj.reshape(-1), 80)
    partials = partials.reshape(NC, partials.shape[0] // NC, d_out)
    return _combine(partials, b.reshape(1, d_out), n_nodes)


# speedup vs baseline: 13.0405x; 1.0565x over previous
import functools

import jax
import jax.numpy as jnp
from jax import lax
from jax.experimental import pallas as pl
from jax.experimental.pallas import tpu as pltpu
from jax.experimental.pallas import tpu_sc as plsc

NC = 2
NS = 16
NW = NC * NS
LANES = 16


def _combine_matmul(partials, w, bias, n_nodes):
    d_in = partials.shape[2]
    d_out = w.shape[1]
    bm = 1000

    def body(p_ref, w_ref, b_ref, o_ref):
        o_ref[...] = jnp.dot(p_ref[0] + p_ref[1], w_ref[...],
                             preferred_element_type=jnp.float32) + b_ref[...]

    return pl.pallas_call(
        body,
        grid=(n_nodes // bm,),
        in_specs=[pl.BlockSpec((2, bm, d_in), lambda i: (0, i, 0)),
                  pl.BlockSpec((d_in, d_out), lambda i: (0, 0)),
                  pl.BlockSpec((1, d_out), lambda i: (0, 0))],
        out_specs=pl.BlockSpec((bm, d_out), lambda i: (i, 0)),
        out_shape=jax.ShapeDtypeStruct((n_nodes, d_out), jnp.float32),
    )(partials, w, bias)


def _sc_aggregate(support, adj1, chunk):
    n_nodes, d = support.shape
    n_edges = adj1.shape[0] // 2
    epw = n_edges // NW
    nchunk = epw // chunk
    rpt = -(-n_nodes // (NS * 8)) * 8
    n_pad = rpt * NS

    mesh = plsc.VectorSubcoreMesh(core_axis_name="c", subcore_axis_name="s",
                                  num_cores=NC, num_subcores=NS)

    @functools.partial(
        pl.kernel,
        out_type=jax.ShapeDtypeStruct((NC * n_pad, d), jnp.float32),
        mesh=mesh,
        scratch_types=[
            pltpu.VMEM_SHARED((n_pad, d), jnp.float32),
            pltpu.VMEM((epw,), jnp.int32),
            pltpu.VMEM((chunk,), jnp.int32),
            pltpu.VMEM((chunk,), jnp.int32),
            pltpu.VMEM((chunk, d), jnp.float32),
            pltpu.VMEM((chunk, d), jnp.float32),
            pltpu.SemaphoreType.DMA,
            pltpu.SemaphoreType.DMA,
            pltpu.SemaphoreType.DMA,
            pltpu.SemaphoreType.DMA,
            pltpu.SemaphoreType.DMA,
            pltpu.SemaphoreType.DMA,
        ],
    )
    def agg(sup_hbm, adj_hbm, out_hbm, acc, src_all, idxd0, idxd1, rows0,
            rows1, gsem0, gsem1, dsem0, dsem1, ssem0, ssem1):
        c = lax.axis_index("c")
        s = lax.axis_index("s")
        w = c * NS + s
        rows = (rows0, rows1)
        idxd = (idxd0, idxd1)
        gsem = (gsem0, gsem1)
        dsem = (dsem0, dsem1)
        ssem = (ssem0, ssem1)

        pltpu.sync_copy(adj_hbm.at[pl.ds(w * epw, epw)], src_all)
        dst_base = n_edges + w * epw

        zeros = jnp.zeros((LANES,), jnp.float32)
        per_row = d // LANES

        def zstep(i, _):
            rows0[i // per_row, pl.ds((i % per_row) * LANES, LANES)] = zeros
            return 0

        lax.fori_loop(0, chunk * per_row, zstep, 0)
        off_r = 0
        while off_r + chunk <= rpt:
            pltpu.sync_copy(rows0, acc.at[pl.ds(s * rpt + off_r, chunk)])
            off_r += chunk
        if rpt - off_r:
            pltpu.sync_copy(rows0.at[pl.ds(0, rpt - off_r)],
                            acc.at[pl.ds(s * rpt + off_r, rpt - off_r)])
        plsc.subcore_barrier()

        def start_g(i, b):
            pltpu.async_copy(sup_hbm.at[src_all.at[pl.ds(i * chunk, chunk)]],
                             rows[b], gsem[b])

        def wait_g(b):
            pltpu.make_async_copy(sup_hbm.at[src_all.at[pl.ds(0, chunk)]],
                                  rows[b], gsem[b]).wait()

        def start_d(i, b):
            pltpu.async_copy(adj_hbm.at[pl.ds(dst_base + i * chunk, chunk)],
                             idxd[b], dsem[b])

        def wait_d(b):
            pltpu.make_async_copy(adj_hbm.at[pl.ds(dst_base, chunk)],
                                  idxd[b], dsem[b]).wait()

        def scat(b):
            pltpu.async_copy(rows[b], acc.at[idxd[b]], ssem[b],
                             add=True).wait()

        for b in range(2):
            start_g(b, b)
            start_d(b, b)
        n_steady = (nchunk - 3) // 2

        def step(j, _):
            for b in range(2):
                i = 2 * j + b
                wait_g(b)
                wait_d(b)
                scat(b)
                start_g(i + 2, b)
                start_d(i + 2, b)
            return 0

        lax.fori_loop(0, n_steady, step, 0)
        for i in range(2 * n_steady, nchunk):
            b = i % 2
            wait_g(b)
            wait_d(b)
            scat(b)
            if i + 2 < nchunk:
                start_g(i + 2, b)
                start_d(i + 2, b)
        plsc.subcore_barrier()

        pltpu.sync_copy(acc.at[pl.ds(s * rpt, rpt)],
                        out_hbm.at[pl.ds(c * n_pad + s * rpt, rpt)])

    return agg(support, adj1)


def kernel(input, adj, W, b):
    n_nodes, d_in = input.shape
    d_out = W.shape[1]
    partials = _sc_aggregate(input, adj.reshape(-1), 80)
    partials = partials.reshape(NC, partials.shape[0] // NC, d_in)
    return _combine_matmul(partials, W, b.reshape(1, d_out), n_nodes)
